# baseline (speedup 1.0000x reference)
import jax
import jax.numpy as jnp
from jax import lax
from jax.experimental import pallas as pl
from jax.experimental.pallas import tpu as pltpu
from jax.experimental.pallas import tpu_sc as plsc

D = 1024
NC, NS = 2, 16
NW = NC * NS
CH = 16


def _sc_body(emb_hbm, pos_hbm, tim_hbm, flg_hbm, pos_tab_hbm, tim_tab_hbm,
             out_hbm, pos_idx, tim_idx, flg_v, pos_buf, tim_buf, emb_buf,
             sem_i, sem_f, sem_g, sem_w):
    B, S = pos_hbm.shape
    T = B * S
    tpw = T // NW
    wps = S // tpw
    wid = lax.axis_index("s") * NC + lax.axis_index("c")
    b = wid // wps
    s0 = (wid % wps) * tpw

    ci0 = pltpu.async_copy(pos_hbm.at[b, pl.ds(s0, tpw)], pos_idx, sem_i)
    ci1 = pltpu.async_copy(tim_hbm.at[b, pl.ds(s0, tpw)], tim_idx, sem_i)
    ci2 = pltpu.async_copy(flg_hbm.at[b, pl.ds(s0, tpw)], flg_v, sem_i)
    ci0.wait()
    ci1.wait()
    ci2.wait()
    cf = pltpu.async_copy(flg_v,
                          out_hbm.at[b, pl.ds(s0, tpw), pl.ds(3 * D, 1)],
                          sem_f)

    def chunk(i, _):
        s = s0 + i * CH
        off = i * CH
        gp = pltpu.async_copy(
            pos_tab_hbm.at[pos_idx.at[pl.ds(off, CH)]], pos_buf, sem_g)
        gt = pltpu.async_copy(
            tim_tab_hbm.at[tim_idx.at[pl.ds(off, CH)]], tim_buf, sem_g)
        ge = pltpu.async_copy(emb_hbm.at[b, pl.ds(s, CH)], emb_buf, sem_g)
        gp.wait()
        gt.wait()
        ge.wait()
        we = pltpu.async_copy(
            emb_buf, out_hbm.at[b, pl.ds(s, CH), pl.ds(0, D)], sem_w)
        wp = pltpu.async_copy(
            pos_buf, out_hbm.at[b, pl.ds(s, CH), pl.ds(D, D)], sem_w)
        wt = pltpu.async_copy(
            tim_buf, out_hbm.at[b, pl.ds(s, CH), pl.ds(2 * D, D)], sem_w)
        we.wait()
        wp.wait()
        wt.wait()
        return ()

    lax.fori_loop(0, tpw // CH, chunk, ())
    cf.wait()


def kernel(embeddings, positions, times, source_flags, pos_table, time_table):
    B, S, Dm = embeddings.shape
    T = B * S
    tpw = T // NW
    pos = positions.astype(jnp.int32)
    tim = times.astype(jnp.int32)
    flg = source_flags[..., None].astype(jnp.float32)
    mesh = plsc.VectorSubcoreMesh(
        core_axis_name="c", subcore_axis_name="s",
        num_cores=NC, num_subcores=NS)
    out = pl.kernel(
        _sc_body,
        out_type=jax.ShapeDtypeStruct((B, S, 3 * Dm + 1), jnp.float32),
        mesh=mesh,
        scratch_types=[
            pltpu.VMEM((tpw,), jnp.int32),
            pltpu.VMEM((tpw,), jnp.int32),
            pltpu.VMEM((tpw, 1), jnp.float32),
            pltpu.VMEM((CH, Dm), jnp.float32),
            pltpu.VMEM((CH, Dm), jnp.float32),
            pltpu.VMEM((CH, Dm), jnp.float32),
            pltpu.SemaphoreType.DMA,
            pltpu.SemaphoreType.DMA,
            pltpu.SemaphoreType.DMA,
            pltpu.SemaphoreType.DMA,
        ],
    )(embeddings, pos, tim, flg, pos_table, time_table)
    return out


# speedup vs baseline: 1.0033x; 1.0033x over previous
import jax
import jax.numpy as jnp
from jax import lax
from jax.experimental import pallas as pl
from jax.experimental.pallas import tpu as pltpu
from jax.experimental.pallas import tpu_sc as plsc

D = 1024
NC, NS = 2, 16
NW = NC * NS
CH = 16


def _sc_body(emb_hbm, pos_hbm, tim_hbm, flg_hbm, pos_tab_hbm, tim_tab_hbm,
             out_hbm, pos_idx, tim_idx, flg_v, pos_buf, tim_buf, emb_buf,
             sem_i, sem_f, sem_g, sem_w):
    B, S = pos_hbm.shape
    T = B * S
    tpw = T // NW
    wps = S // tpw
    wid = lax.axis_index("c") * NS + lax.axis_index("s")
    b = wid // wps
    s0 = (wid % wps) * tpw

    ci0 = pltpu.async_copy(pos_hbm.at[b, pl.ds(s0, tpw)], pos_idx, sem_i)
    ci1 = pltpu.async_copy(tim_hbm.at[b, pl.ds(s0, tpw)], tim_idx, sem_i)
    ci2 = pltpu.async_copy(flg_hbm.at[b, pl.ds(s0, tpw)], flg_v, sem_i)
    ci0.wait()
    ci1.wait()
    ci2.wait()
    cf = pltpu.async_copy(flg_v,
                          out_hbm.at[b, pl.ds(s0, tpw), pl.ds(3 * D, 1)],
                          sem_f)

    def chunk(i, _):
        s = s0 + i * CH
        off = i * CH
        gp = pltpu.async_copy(
            pos_tab_hbm.at[pos_idx.at[pl.ds(off, CH)]], pos_buf, sem_g)
        gt = pltpu.async_copy(
            tim_tab_hbm.at[tim_idx.at[pl.ds(off, CH)]], tim_buf, sem_g)
        ge = pltpu.async_copy(emb_hbm.at[b, pl.ds(s, CH)], emb_buf, sem_g)
        gp.wait()
        gt.wait()
        ge.wait()
        we = pltpu.async_copy(
            emb_buf, out_hbm.at[b, pl.ds(s, CH), pl.ds(0, D)], sem_w)
        wp = pltpu.async_copy(
            pos_buf, out_hbm.at[b, pl.ds(s, CH), pl.ds(D, D)], sem_w)
        wt = pltpu.async_copy(
            tim_buf, out_hbm.at[b, pl.ds(s, CH), pl.ds(2 * D, D)], sem_w)
        we.wait()
        wp.wait()
        wt.wait()
        return ()

    lax.fori_loop(0, tpw // CH, chunk, ())
    cf.wait()


def kernel(embeddings, positions, times, source_flags, pos_table, time_table):
    B, S, Dm = embeddings.shape
    T = B * S
    tpw = T // NW
    pos = positions.astype(jnp.int32)
    tim = times.astype(jnp.int32)
    flg = source_flags[..., None].astype(jnp.float32)
    mesh = plsc.VectorSubcoreMesh(
        core_axis_name="c", subcore_axis_name="s",
        num_cores=NC, num_subcores=NS)
    out = pl.kernel(
        _sc_body,
        out_type=jax.ShapeDtypeStruct((B, S, 3 * Dm + 1), jnp.float32),
        mesh=mesh,
        scratch_types=[
            pltpu.VMEM((tpw,), jnp.int32),
            pltpu.VMEM((tpw,), jnp.int32),
            pltpu.VMEM((tpw, 1), jnp.float32),
            pltpu.VMEM((CH, Dm), jnp.float32),
            pltpu.VMEM((CH, Dm), jnp.float32),
            pltpu.VMEM((CH, Dm), jnp.float32),
            pltpu.SemaphoreType.DMA,
            pltpu.SemaphoreType.DMA,
            pltpu.SemaphoreType.DMA,
            pltpu.SemaphoreType.DMA,
        ],
    )(embeddings, pos, tim, flg, pos_table, time_table)
    return out
